# Initial kernel scaffold; baseline (speedup 1.0000x reference)
import jax
import jax.numpy as jnp
from jax.experimental import pallas as pl


def kernel(x, adj_nodes, adj_offset, degrees, choices):
    raise NotImplementedError("write your pallas kernel here")


# speedup vs baseline: 1.0056x; 1.0056x over previous
import jax
import jax.numpy as jnp
from jax import lax
from jax.experimental import pallas as pl
from jax.experimental.pallas import tpu as pltpu
from jax.experimental.pallas import tpu_sc as plsc

STEPS = 16
L = STEPS + 1
WIN = 8
N = 100000
DEG = 16
E = N * DEG
D = 128

NT = 32
W = 3200
NP = NT * W
CK = 64
CH = W // CK


def _mesh():
    return plsc.VectorSubcoreMesh(core_axis_name="c", subcore_axis_name="s",
                                  num_cores=2, num_subcores=16)


def _tile_base():
    cid = lax.axis_index("c")
    sid = lax.axis_index("s")
    return (sid * 2 + cid) * W


def _sc_walk_body(adjn_h, adjoff_h, deg_h, cho_h,
                  walks_h,
                  wk, degb, offb, chb, altb, newb, anewb,
                  gsem, wsem):
    base = _tile_base()

    def init_c(c, carry):
        for o in range(CK // 16):
            ids = base + c * CK + o * 16 + lax.iota(jnp.int32, 16)
            ids = jnp.minimum(ids, N - 1)
            wk[0, pl.ds(c * CK + o * 16, 16)] = ids
        return carry
    lax.fori_loop(0, CH, init_c, 0)

    def step(i, carry, with_bt):
        iprev = jnp.maximum(i - 1, 0)

        def cho_idx(c, carry):
            for o in range(CK // 16):
                sl = pl.ds(c * CK + o * 16, 16)
                chb[sl] = wk[0, sl] + i * N
            return carry
        lax.fori_loop(0, CH, cho_idx, 0)

        def fire1(c, carry):
            ck = pl.ds(c * CK, CK)
            pltpu.async_copy(deg_h.at[wk.at[i, ck]], degb.at[ck], gsem)
            pltpu.async_copy(adjoff_h.at[wk.at[i, ck]], offb.at[ck], gsem)
            pltpu.async_copy(cho_h.at[chb.at[ck]], newb.at[ck], gsem)
            return carry
        lax.fori_loop(0, CH, fire1, 0)

        def drain1(c, carry):
            ck = pl.ds(c * CK, CK)
            pltpu.make_async_copy(deg_h.at[wk.at[i, ck]],
                                  degb.at[ck], gsem).wait()
            pltpu.make_async_copy(adjoff_h.at[wk.at[i, ck]],
                                  offb.at[ck], gsem).wait()
            pltpu.make_async_copy(cho_h.at[chb.at[ck]],
                                  newb.at[ck], gsem).wait()
            return carry
        lax.fori_loop(0, CH, drain1, 0)

        def comp1(c, carry):
            for o in range(CK // 16):
                sl = pl.ds(c * CK + o * 16, 16)
                d = degb[sl]
                off = offb[sl]
                ch = newb[sl]
                ei = lax.rem(ch, d)
                nbd = jnp.maximum(d - 1, 1)
                ai = lax.rem(ei + 1 + lax.rem(ch, nbd), d)
                chb[sl] = off + ei
                altb[sl] = off + ai
            return carry
        lax.fori_loop(0, CH, comp1, 0)

        def fire2(c, carry):
            ck = pl.ds(c * CK, CK)
            pltpu.async_copy(adjn_h.at[chb.at[ck]], newb.at[ck], gsem)
            if with_bt:
                pltpu.async_copy(adjn_h.at[altb.at[ck]], anewb.at[ck], gsem)
            return carry
        lax.fori_loop(0, CH, fire2, 0)

        def drain2(c, carry):
            ck = pl.ds(c * CK, CK)
            pltpu.make_async_copy(adjn_h.at[chb.at[ck]],
                                  newb.at[ck], gsem).wait()
            if with_bt:
                pltpu.make_async_copy(adjn_h.at[altb.at[ck]],
                                      anewb.at[ck], gsem).wait()
            return carry
        lax.fori_loop(0, CH, drain2, 0)

        def comp2(c, carry):
            for o in range(CK // 16):
                sl = pl.ds(c * CK + o * 16, 16)
                nw = newb[sl]
                if with_bt:
                    an = anewb[sl]
                    pv = wk[iprev, sl]
                    wk[i + 1, sl] = jnp.where(nw == pv, an, nw)
                else:
                    wk[i + 1, sl] = nw
            return carry
        lax.fori_loop(0, CH, comp2, 0)
        return carry
    step(0, 0, with_bt=False)
    lax.fori_loop(1, STEPS, lambda i, c: step(i, c, with_bt=True), 0)

    def wout_fire(j, carry):
        pltpu.async_copy(wk.at[j], walks_h.at[pl.ds(j * NP + base, W)], wsem)
        return carry
    lax.fori_loop(0, L, wout_fire, 0)

    def wout_drain(j, carry):
        pltpu.make_async_copy(wk.at[j],
                              walks_h.at[pl.ds(j * NP + base, W)],
                              wsem).wait()
        return carry
    lax.fori_loop(0, L, wout_drain, 0)


def _sc_pool_body(x_h, walks_h,
                  pooled_h,
                  wk, acc, ring,
                  lsem, rsem0, rsem1, psem):
    base = _tile_base()

    def lin(j, carry):
        pltpu.async_copy(walks_h.at[pl.ds(j * NP + base, W)], wk.at[j], lsem)
        return carry
    lax.fori_loop(0, L, lin, 0)

    def lwait(j, carry):
        pltpu.make_async_copy(walks_h.at[pl.ds(j * NP + base, W)],
                              wk.at[j], lsem).wait()
        return carry
    lax.fori_loop(0, L, lwait, 0)

    inv = jnp.float32(1.0 / L)

    def chunkloop(c, carry):
        ck0 = pl.ds(c * CK, CK)
        pltpu.async_copy(x_h.at[wk.at[0, ck0]], ring.at[0], rsem0)
        for j in range(L):
            b = j % 2
            sem = rsem0 if b == 0 else rsem1
            pltpu.make_async_copy(x_h.at[wk.at[j, ck0]],
                                  ring.at[b], sem).wait()
            if j + 1 < L:
                nsem = rsem1 if b == 0 else rsem0
                pltpu.async_copy(x_h.at[wk.at[j + 1, ck0]],
                                 ring.at[1 - b], nsem)

            def accrow(r, carry):
                for o in range(D // 16):
                    so = pl.ds(o * 16, 16)
                    if j == 0:
                        acc[r, so] = ring[b, r, so]
                    elif j == L - 1:
                        acc[r, so] = (acc[r, so] + ring[b, r, so]) * inv
                    else:
                        acc[r, so] = acc[r, so] + ring[b, r, so]
                return carry
            lax.fori_loop(0, CK, accrow, 0)
        cpw = pltpu.async_copy(acc, pooled_h.at[pl.ds(base + c * CK, CK), :],
                               psem)
        cpw.wait()
        return carry
    lax.fori_loop(0, CH, chunkloop, 0)


def _sc_walk(adj_nodes, adj_offset, degrees, choices_flat):
    kfn = pl.kernel(
        _sc_walk_body,
        out_type=[jax.ShapeDtypeStruct((L * NP,), jnp.int32)],
        mesh=_mesh(),
        scratch_types=[
            pltpu.VMEM((L, W), jnp.int32),
            pltpu.VMEM((W,), jnp.int32),
            pltpu.VMEM((W,), jnp.int32),
            pltpu.VMEM((W,), jnp.int32),
            pltpu.VMEM((W,), jnp.int32),
            pltpu.VMEM((W,), jnp.int32),
            pltpu.VMEM((W,), jnp.int32),
            pltpu.SemaphoreType.DMA,
            pltpu.SemaphoreType.DMA,
        ],
    )
    return kfn(adj_nodes, adj_offset, degrees, choices_flat)[0]


def _sc_pool(x, walks_flat):
    kfn = pl.kernel(
        _sc_pool_body,
        out_type=[jax.ShapeDtypeStruct((NP, D), jnp.float32)],
        mesh=_mesh(),
        scratch_types=[
            pltpu.VMEM((L, W), jnp.int32),
            pltpu.VMEM((CK, D), jnp.float32),
            pltpu.VMEM((2, CK, D), jnp.float32),
            pltpu.SemaphoreType.DMA,
            pltpu.SemaphoreType.DMA,
            pltpu.SemaphoreType.DMA,
            pltpu.SemaphoreType.DMA,
        ],
    )
    return kfn(x, walks_flat)[0]


def _idenc_body(w_ref, o_ref):
    w = w_ref[...]
    bn = w.shape[1]
    rows = []
    for t in range(WIN):
        d = WIN - t
        eq = (w[d:, :] == w[:-d, :]).astype(jnp.int32)
        z = jnp.zeros((d, bn), jnp.int32)
        rows.append(jnp.concatenate([z, eq], axis=0))
    o_ref[...] = jnp.stack(rows, axis=1) != 0


def _idenc_tc(walks_pad):
    BN = 2048
    grid = (NP // BN,)
    return pl.pallas_call(
        _idenc_body,
        grid=grid,
        in_specs=[pl.BlockSpec((L, BN), lambda i: (0, i))],
        out_specs=pl.BlockSpec((L, WIN, BN), lambda i: (0, 0, i)),
        out_shape=jax.ShapeDtypeStruct((L, WIN, NP), jnp.bool_),
    )(walks_pad)


def kernel(x, adj_nodes, adj_offset, degrees, choices):
    walks_flat = _sc_walk(adj_nodes, adj_offset, degrees,
                          choices.reshape(-1))
    pooled_pad = _sc_pool(x, walks_flat)
    walks_pad = walks_flat.reshape(L, NP)
    walks = walks_pad[:, :N]
    pooled = pooled_pad[:N]
    id_enc = _idenc_tc(walks_pad)[:, :, :N]
    return pooled, walks, id_enc


# speedup vs baseline: 1.2609x; 1.2539x over previous
import jax
import jax.numpy as jnp
from jax import lax
from jax.experimental import pallas as pl
from jax.experimental.pallas import tpu as pltpu
from jax.experimental.pallas import tpu_sc as plsc

STEPS = 16
L = STEPS + 1
WIN = 8
N = 100000
DEG = 16
E = N * DEG
D = 128

NT = 32
W = 3200
NP = NT * W
CK = 64
CH = W // CK
RING = 4


def _mesh():
    return plsc.VectorSubcoreMesh(core_axis_name="c", subcore_axis_name="s",
                                  num_cores=2, num_subcores=16)


def _tile_base():
    cid = lax.axis_index("c")
    sid = lax.axis_index("s")
    return (sid * 2 + cid) * W


def _sc_walk_body(adjn_h, adjoff_h, deg_h, cho_h,
                  walks_h,
                  startb, rowa, rowb, degb, offb, chb, altb, newb, anewb,
                  gsem, wsem):
    base = _tile_base()

    def init_v(v, carry):
        ids = base + v * 16 + lax.iota(jnp.int32, 16)
        ids = jnp.minimum(ids, N - 1)
        startb[pl.ds(v * 16, 16)] = ids
        return carry
    lax.fori_loop(0, W // 16, init_v, 0)
    pltpu.async_copy(startb, walks_h.at[pl.ds(base, W)], wsem)

    def do_step(i, cur, prev, dst, with_bt, wait_dst_row):
        def cho_idx(v, carry):
            sl = pl.ds(v * 16, 16)
            chb[sl] = startb[sl] + i * N
            return carry
        lax.fori_loop(0, W // 16, cho_idx, 0)

        pltpu.async_copy(deg_h.at[cur], degb, gsem)
        pltpu.async_copy(adjoff_h.at[cur], offb, gsem)
        pltpu.async_copy(cho_h.at[chb], newb, gsem)
        pltpu.make_async_copy(deg_h.at[cur], degb, gsem).wait()
        pltpu.make_async_copy(adjoff_h.at[cur], offb, gsem).wait()
        pltpu.make_async_copy(cho_h.at[chb], newb, gsem).wait()

        def comp1(v, carry):
            sl = pl.ds(v * 16, 16)
            d = degb[sl]
            off = offb[sl]
            ch = newb[sl]
            ei = lax.rem(ch, d)
            nbd = jnp.maximum(d - 1, 1)
            ai = lax.rem(ei + 1 + lax.rem(ch, nbd), d)
            chb[sl] = off + ei
            altb[sl] = off + ai
            return carry
        lax.fori_loop(0, W // 16, comp1, 0)

        pltpu.async_copy(adjn_h.at[chb], newb, gsem)
        if with_bt:
            pltpu.async_copy(adjn_h.at[altb], anewb, gsem)
        pltpu.make_async_copy(adjn_h.at[chb], newb, gsem).wait()
        if with_bt:
            pltpu.make_async_copy(adjn_h.at[altb], anewb, gsem).wait()

        if wait_dst_row:
            pltpu.make_async_copy(
                dst, walks_h.at[pl.ds((i - 1) * NP + base, W)], wsem).wait()

        def comp2(v, carry):
            sl = pl.ds(v * 16, 16)
            nw = newb[sl]
            if with_bt:
                an = anewb[sl]
                pv = prev[sl]
                dst[sl] = jnp.where(nw == pv, an, nw)
            else:
                dst[sl] = nw
            return carry
        lax.fori_loop(0, W // 16, comp2, 0)

        pltpu.async_copy(dst, walks_h.at[pl.ds((i + 1) * NP + base, W)],
                         wsem)

    do_step(0, startb, None, rowb, False, False)
    do_step(1, rowb, startb, rowa, True, False)

    def pair(t, carry):
        do_step(2 * t, rowa, rowb, rowb, True, True)
        do_step(2 * t + 1, rowb, rowa, rowa, True, True)
        return carry
    lax.fori_loop(1, STEPS // 2, pair, 0)

    pltpu.make_async_copy(startb, walks_h.at[pl.ds(base, W)], wsem).wait()
    pltpu.make_async_copy(rowb, walks_h.at[pl.ds(15 * NP + base, W)],
                          wsem).wait()
    pltpu.make_async_copy(rowa, walks_h.at[pl.ds(16 * NP + base, W)],
                          wsem).wait()


def _sc_pool_body(x_h, walks_h,
                  pooled_h,
                  wk, acc, ring,
                  lsem, rs0, rs1, rs2, rs3, psem):
    base = _tile_base()
    sems = [rs0, rs1, rs2, rs3]

    def lin(j, carry):
        pltpu.async_copy(walks_h.at[pl.ds(j * NP + base, W)], wk.at[j], lsem)
        return carry
    lax.fori_loop(0, L, lin, 0)

    def lwait(j, carry):
        pltpu.make_async_copy(walks_h.at[pl.ds(j * NP + base, W)],
                              wk.at[j], lsem).wait()
        return carry
    lax.fori_loop(0, L, lwait, 0)

    inv = jnp.float32(1.0 / L)

    def chunkloop(c, carry):
        ck0 = pl.ds(c * CK, CK)
        for j in range(RING - 1):
            pltpu.async_copy(x_h.at[wk.at[j, ck0]], ring.at[j % RING],
                             sems[j % RING])
        for j in range(L):
            b = j % RING
            pltpu.make_async_copy(x_h.at[wk.at[j, ck0]],
                                  ring.at[b], sems[b]).wait()
            jn = j + RING - 1
            if jn < L:
                pltpu.async_copy(x_h.at[wk.at[jn, ck0]],
                                 ring.at[jn % RING], sems[jn % RING])

            def accrow(r, carry):
                for o in range(D // 16):
                    so = pl.ds(o * 16, 16)
                    if j == 0:
                        acc[r, so] = ring[b, r, so]
                    elif j == L - 1:
                        acc[r, so] = (acc[r, so] + ring[b, r, so]) * inv
                    else:
                        acc[r, so] = acc[r, so] + ring[b, r, so]
                return carry
            lax.fori_loop(0, CK, accrow, 0)
        cpw = pltpu.async_copy(acc, pooled_h.at[pl.ds(base + c * CK, CK), :],
                               psem)
        cpw.wait()
        return carry
    lax.fori_loop(0, CH, chunkloop, 0)


def _sc_walk(adj_nodes, adj_offset, degrees, choices_flat):
    kfn = pl.kernel(
        _sc_walk_body,
        out_type=[jax.ShapeDtypeStruct((L * NP,), jnp.int32)],
        mesh=_mesh(),
        scratch_types=[
            pltpu.VMEM((W,), jnp.int32),
            pltpu.VMEM((W,), jnp.int32),
            pltpu.VMEM((W,), jnp.int32),
            pltpu.VMEM((W,), jnp.int32),
            pltpu.VMEM((W,), jnp.int32),
            pltpu.VMEM((W,), jnp.int32),
            pltpu.VMEM((W,), jnp.int32),
            pltpu.VMEM((W,), jnp.int32),
            pltpu.VMEM((W,), jnp.int32),
            pltpu.SemaphoreType.DMA,
            pltpu.SemaphoreType.DMA,
        ],
    )
    return kfn(adj_nodes, adj_offset, degrees, choices_flat)[0]


def _sc_pool(x, walks_flat):
    kfn = pl.kernel(
        _sc_pool_body,
        out_type=[jax.ShapeDtypeStruct((NP, D), jnp.float32)],
        mesh=_mesh(),
        scratch_types=[
            pltpu.VMEM((L, W), jnp.int32),
            pltpu.VMEM((CK, D), jnp.float32),
            pltpu.VMEM((RING, CK, D), jnp.float32),
            pltpu.SemaphoreType.DMA,
            pltpu.SemaphoreType.DMA,
            pltpu.SemaphoreType.DMA,
            pltpu.SemaphoreType.DMA,
            pltpu.SemaphoreType.DMA,
            pltpu.SemaphoreType.DMA,
        ],
    )
    return kfn(x, walks_flat)[0]


def _idenc_body(w_ref, o_ref):
    w = w_ref[...]
    bn = w.shape[1]
    rows = []
    for t in range(WIN):
        d = WIN - t
        eq = (w[d:, :] == w[:-d, :]).astype(jnp.int32)
        z = jnp.zeros((d, bn), jnp.int32)
        rows.append(jnp.concatenate([z, eq], axis=0))
    o_ref[...] = jnp.stack(rows, axis=1) != 0


def _idenc_tc(walks_pad):
    BN = 2048
    grid = (NP // BN,)
    return pl.pallas_call(
        _idenc_body,
        grid=grid,
        in_specs=[pl.BlockSpec((L, BN), lambda i: (0, i))],
        out_specs=pl.BlockSpec((L, WIN, BN), lambda i: (0, 0, i)),
        out_shape=jax.ShapeDtypeStruct((L, WIN, NP), jnp.bool_),
    )(walks_pad)


def kernel(x, adj_nodes, adj_offset, degrees, choices):
    walks_flat = _sc_walk(adj_nodes, adj_offset, degrees,
                          choices.reshape(-1))
    pooled_pad = _sc_pool(x, walks_flat)
    walks_pad = walks_flat.reshape(L, NP)
    walks = walks_pad[:, :N]
    pooled = pooled_pad[:N]
    id_enc = _idenc_tc(walks_pad)[:, :, :N]
    return pooled, walks, id_enc
